# Initial kernel scaffold; baseline (speedup 1.0000x reference)
import jax
import jax.numpy as jnp
from jax.experimental import pallas as pl


def kernel(x, edge_index, edge_attr, W1_l, b1, W1_r, W2_l, b2, W2_r):
    raise NotImplementedError("write your pallas kernel here")


# speedup vs baseline: 5.3257x; 5.3257x over previous
import functools

import jax
import jax.numpy as jnp
from jax import lax
from jax.experimental import pallas as pl
from jax.experimental.pallas import tpu as pltpu
from jax.experimental.pallas import tpu_sc as plsc

N_NODES = 10000
N_EDGES = 160000
D_FULL = 256
HALF = 128
CW = 16

NTEC = 16
EPT = N_EDGES // NTEC
K = 80
CHUNKS = EPT // K
N_PAD = 10240
RPT = N_PAD // NTEC
ZR = 32

_mesh = plsc.VectorSubcoreMesh(core_axis_name="c", subcore_axis_name="s")


def _make_sc_agg():
  out_type = [jax.ShapeDtypeStruct((2, N_PAD, HALF), jnp.float32)]
  scratch_types = [
      pltpu.VMEM((CHUNKS, K), jnp.int32),
      pltpu.VMEM((CHUNKS, K), jnp.int32),
      pltpu.VMEM((K, HALF), jnp.float32),
      pltpu.VMEM((ZR, HALF), jnp.float32),
      pltpu.VMEM_SHARED((N_PAD, HALF), jnp.float32),
      pltpu.SemaphoreType.DMA,
  ]

  @functools.partial(
      pl.kernel, mesh=_mesh, out_type=out_type, scratch_types=scratch_types,
      compiler_params=pltpu.CompilerParams(use_tc_tiling_on_sc=False))
  def sc_agg(xcat, src, dst, agg_out, src_v, dst_v, rows_v, zbuf, acc_sh, sem):
    c = lax.axis_index("c")
    s = lax.axis_index("s")
    r0 = s * RPT

    zero16 = jnp.zeros((16,), jnp.float32)

    pltpu.sync_copy(src.at[c, s], src_v)
    pltpu.sync_copy(dst.at[s], dst_v)

    def zb(i, _):
      zbuf[i // 8, pl.ds((i % 8) * 16, 16)] = zero16
      return 0
    lax.fori_loop(0, ZR * 8, zb, 0)
    for z in range(RPT // ZR):
      pltpu.sync_copy(zbuf, acc_sh.at[pl.ds(r0 + z * ZR, ZR)])

    plsc.subcore_barrier()

    def chunk(j, _):
      pltpu.async_copy(xcat.at[src_v.at[j]], rows_v, sem).wait()
      pltpu.sync_copy(rows_v, acc_sh.at[dst_v.at[j]], add=True)
      return 0
    lax.fori_loop(0, CHUNKS, chunk, 0)

    plsc.subcore_barrier()

    pltpu.sync_copy(acc_sh.at[pl.ds(r0, RPT)], agg_out.at[c, pl.ds(r0, RPT)])

  return sc_agg


def _make_sc_cnt():
  out_type = [jax.ShapeDtypeStruct((N_PAD, CW), jnp.float32)]
  scratch_types = [
      pltpu.VMEM((CHUNKS, K), jnp.int32),
      pltpu.VMEM((K, CW), jnp.float32),
      pltpu.VMEM((ZR, CW), jnp.float32),
      pltpu.VMEM_SHARED((N_PAD, CW), jnp.float32),
  ]

  @functools.partial(
      pl.kernel, mesh=_mesh, out_type=out_type, scratch_types=scratch_types,
      compiler_params=pltpu.CompilerParams(use_tc_tiling_on_sc=False))
  def sc_cnt(dst, cnt_out, dst_v, ones_v, zcnt, cnt_sh):
    c = lax.axis_index("c")
    s = lax.axis_index("s")
    r0 = s * RPT

    @pl.when(c == 0)
    def _():
      zero16 = jnp.zeros((16,), jnp.float32)
      one16 = jnp.ones((16,), jnp.float32)
      pltpu.sync_copy(dst.at[s], dst_v)

      def zc(i, _):
        zcnt[i, :] = zero16
        return 0
      lax.fori_loop(0, ZR, zc, 0)
      def ob(i, _):
        ones_v[i, :] = one16
        return 0
      lax.fori_loop(0, K, ob, 0)
      for z in range(RPT // ZR):
        pltpu.sync_copy(zcnt, cnt_sh.at[pl.ds(r0 + z * ZR, ZR)])

      plsc.subcore_barrier()

      def chunk(j, _):
        pltpu.sync_copy(ones_v, cnt_sh.at[dst_v.at[j]], add=True)
        return 0
      lax.fori_loop(0, CHUNKS, chunk, 0)

      plsc.subcore_barrier()
      pltpu.sync_copy(cnt_sh.at[pl.ds(r0, RPT)], cnt_out.at[pl.ds(r0, RPT)])

  return sc_cnt


_sc_agg = _make_sc_agg()
_sc_cnt = _make_sc_cnt()

R_BLK = 1000
NB = N_NODES // R_BLK


def _make_tc_layer(relu, split_out):
  def body(alo, ahi, cntr, xlo, xhi, wll, wlh, wrl, wrh, br, *outs):
    inv = 1.0 / jnp.maximum(cntr[:, 0:1], 1.0)
    acc = jnp.dot(alo[0] * inv, wll[...], preferred_element_type=jnp.float32)
    acc += jnp.dot(ahi[0] * inv, wlh[...], preferred_element_type=jnp.float32)
    acc += jnp.dot(xlo[...], wrl[...], preferred_element_type=jnp.float32)
    acc += jnp.dot(xhi[...], wrh[...], preferred_element_type=jnp.float32)
    acc += br[...]
    if relu:
      acc = jnp.maximum(acc, 0.0)
    if split_out:
      outs[0][...] = acc[:, :HALF]
      outs[1][...] = acc[:, HALF:]
    else:
      outs[0][...] = acc

  w_spec = pl.BlockSpec((HALF, D_FULL), lambda i: (0, 0))
  in_specs = [
      pl.BlockSpec((1, R_BLK, HALF), lambda i: (0, i, 0)),
      pl.BlockSpec((1, R_BLK, HALF), lambda i: (1, i, 0)),
      pl.BlockSpec((R_BLK, CW), lambda i: (i, 0)),
      pl.BlockSpec((R_BLK, HALF), lambda i: (i, 0)),
      pl.BlockSpec((R_BLK, HALF), lambda i: (i + NB, 0)),
      w_spec, w_spec, w_spec, w_spec,
      pl.BlockSpec((1, D_FULL), lambda i: (0, 0)),
  ]
  if split_out:
    out_specs = [pl.BlockSpec((R_BLK, HALF), lambda i: (i, 0))] * 2
    out_shape = [jax.ShapeDtypeStruct((N_NODES, HALF), jnp.float32)] * 2
  else:
    out_specs = pl.BlockSpec((R_BLK, D_FULL), lambda i: (i, 0))
    out_shape = jax.ShapeDtypeStruct((N_NODES, D_FULL), jnp.float32)
  return pl.pallas_call(body, grid=(NB,), in_specs=in_specs,
                        out_specs=out_specs, out_shape=out_shape)


_tc_layer1 = _make_tc_layer(relu=True, split_out=True)
_tc_layer2 = _make_tc_layer(relu=False, split_out=False)


def _unwrap(r):
  return r[0] if isinstance(r, (tuple, list)) else r


def kernel(x, edge_index, edge_attr, W1_l, b1, W1_r, W2_l, b2, W2_r):
  del edge_attr
  src = edge_index[0].astype(jnp.int32)
  dst = edge_index[1].astype(jnp.int32).reshape(NTEC, CHUNKS, K)
  src2 = jnp.stack([src, src + N_NODES]).reshape(2, NTEC, CHUNKS, K)

  xcat = jnp.concatenate([x[:, :HALF], x[:, HALF:]], axis=0)

  agg1 = _unwrap(_sc_agg(xcat, src2, dst))
  cnt = _unwrap(_sc_cnt(dst))

  w1ll, w1lh = W1_l.T[:HALF], W1_l.T[HALF:]
  w1rl, w1rh = W1_r.T[:HALF], W1_r.T[HALF:]
  h_lo, h_hi = _tc_layer1(agg1, agg1, cnt, xcat, xcat, w1ll, w1lh, w1rl,
                          w1rh, b1.reshape(1, D_FULL))

  hcat = jnp.concatenate([h_lo, h_hi], axis=0)
  agg2 = _unwrap(_sc_agg(hcat, src2, dst))

  w2ll, w2lh = W2_l.T[:HALF], W2_l.T[HALF:]
  w2rl, w2rh = W2_r.T[:HALF], W2_r.T[HALF:]
  out = _tc_layer2(agg2, agg2, cnt, hcat, hcat, w2ll, w2lh, w2rl, w2rh,
                   b2.reshape(1, D_FULL))
  return out


# speedup vs baseline: 8.3893x; 1.5752x over previous
import functools

import jax
import jax.numpy as jnp
from jax import lax
from jax.experimental import pallas as pl
from jax.experimental.pallas import tpu as pltpu
from jax.experimental.pallas import tpu_sc as plsc

N_NODES = 10000
N_EDGES = 160000
D_FULL = 256
HALF = 128
CW = 16

NTEC = 16
EPT = N_EDGES // NTEC
K = 100
CHUNKS = EPT // K
N_PAD = 10240
RPT = N_PAD // NTEC
ZR = 8

_mesh = plsc.VectorSubcoreMesh(core_axis_name="c", subcore_axis_name="s")


def _make_sc_agg():
  out_type = [jax.ShapeDtypeStruct((2, N_PAD, HALF), jnp.float32)]
  scratch_types = [
      pltpu.VMEM((CHUNKS, K), jnp.int32),
      pltpu.VMEM((CHUNKS, K), jnp.int32),
      pltpu.VMEM((2, K, HALF), jnp.float32),
      pltpu.VMEM((ZR, HALF), jnp.float32),
      pltpu.VMEM_SHARED((N_PAD, HALF), jnp.float32),
      pltpu.SemaphoreType.DMA,
      pltpu.SemaphoreType.DMA,
  ]

  @functools.partial(
      pl.kernel, mesh=_mesh, out_type=out_type, scratch_types=scratch_types,
      compiler_params=pltpu.CompilerParams(use_tc_tiling_on_sc=False))
  def sc_agg(xcat, src, dst, agg_out, src_v, dst_v, rows_v, zbuf, acc_sh,
             sem0, sem1):
    c = lax.axis_index("c")
    s = lax.axis_index("s")
    r0 = s * RPT

    zero16 = jnp.zeros((16,), jnp.float32)

    pltpu.sync_copy(src.at[c, s], src_v)
    pltpu.sync_copy(dst.at[s], dst_v)

    def zb(i, _):
      zbuf[i // 8, pl.ds((i % 8) * 16, 16)] = zero16
      return 0
    lax.fori_loop(0, ZR * 8, zb, 0)
    for z in range(RPT // ZR):
      pltpu.sync_copy(zbuf, acc_sh.at[pl.ds(r0 + z * ZR, ZR)])

    plsc.subcore_barrier()

    pltpu.async_copy(xcat.at[src_v.at[0]], rows_v.at[0], sem0)
    pltpu.async_copy(xcat.at[src_v.at[1]], rows_v.at[1], sem1)

    def pair(g, _):
      j = 2 * g
      for b, sem in ((0, sem0), (1, sem1)):
        jb = j + b
        pltpu.make_async_copy(xcat.at[src_v.at[jb]], rows_v.at[b], sem).wait()
        pltpu.sync_copy(rows_v.at[b], acc_sh.at[dst_v.at[jb]], add=True)
        @pl.when(jb + 2 < CHUNKS)
        def _():
          pltpu.async_copy(xcat.at[src_v.at[jb + 2]], rows_v.at[b], sem)
      return 0
    lax.fori_loop(0, CHUNKS // 2, pair, 0)

    plsc.subcore_barrier()

    pltpu.sync_copy(acc_sh.at[pl.ds(r0, RPT)], agg_out.at[c, pl.ds(r0, RPT)])

  return sc_agg


def _make_sc_cnt():
  out_type = [jax.ShapeDtypeStruct((N_PAD, CW), jnp.float32)]
  scratch_types = [
      pltpu.VMEM((CHUNKS, K), jnp.int32),
      pltpu.VMEM((K, CW), jnp.float32),
      pltpu.VMEM((ZR, CW), jnp.float32),
      pltpu.VMEM_SHARED((N_PAD, CW), jnp.float32),
  ]

  @functools.partial(
      pl.kernel, mesh=_mesh, out_type=out_type, scratch_types=scratch_types,
      compiler_params=pltpu.CompilerParams(use_tc_tiling_on_sc=False))
  def sc_cnt(dst, cnt_out, dst_v, ones_v, zcnt, cnt_sh):
    c = lax.axis_index("c")
    s = lax.axis_index("s")
    r0 = s * RPT

    @pl.when(c == 0)
    def _():
      zero16 = jnp.zeros((16,), jnp.float32)
      one16 = jnp.ones((16,), jnp.float32)
      pltpu.sync_copy(dst.at[s], dst_v)

      def zc(i, _):
        zcnt[i, :] = zero16
        return 0
      lax.fori_loop(0, ZR, zc, 0)
      def ob(i, _):
        ones_v[i, :] = one16
        return 0
      lax.fori_loop(0, K, ob, 0)
      for z in range(RPT // ZR):
        pltpu.sync_copy(zcnt, cnt_sh.at[pl.ds(r0 + z * ZR, ZR)])

      plsc.subcore_barrier()

      def chunk(j, _):
        pltpu.sync_copy(ones_v, cnt_sh.at[dst_v.at[j]], add=True)
        return 0
      lax.fori_loop(0, CHUNKS, chunk, 0)

      plsc.subcore_barrier()
      pltpu.sync_copy(cnt_sh.at[pl.ds(r0, RPT)], cnt_out.at[pl.ds(r0, RPT)])

  return sc_cnt


_sc_agg = _make_sc_agg()
_sc_cnt = _make_sc_cnt()

R_BLK = 1000
NB = N_NODES // R_BLK


def _make_tc_layer(relu, split_out):
  def body(alo, ahi, cntr, xlo, xhi, wll, wlh, wrl, wrh, br, *outs):
    inv = 1.0 / jnp.maximum(cntr[:, 0:1], 1.0)
    acc = jnp.dot(alo[0] * inv, wll[...], preferred_element_type=jnp.float32)
    acc += jnp.dot(ahi[0] * inv, wlh[...], preferred_element_type=jnp.float32)
    acc += jnp.dot(xlo[...], wrl[...], preferred_element_type=jnp.float32)
    acc += jnp.dot(xhi[...], wrh[...], preferred_element_type=jnp.float32)
    acc += br[...]
    if relu:
      acc = jnp.maximum(acc, 0.0)
    if split_out:
      outs[0][...] = acc[:, :HALF]
      outs[1][...] = acc[:, HALF:]
    else:
      outs[0][...] = acc

  w_spec = pl.BlockSpec((HALF, D_FULL), lambda i: (0, 0))
  in_specs = [
      pl.BlockSpec((1, R_BLK, HALF), lambda i: (0, i, 0)),
      pl.BlockSpec((1, R_BLK, HALF), lambda i: (1, i, 0)),
      pl.BlockSpec((R_BLK, CW), lambda i: (i, 0)),
      pl.BlockSpec((R_BLK, HALF), lambda i: (i, 0)),
      pl.BlockSpec((R_BLK, HALF), lambda i: (i + NB, 0)),
      w_spec, w_spec, w_spec, w_spec,
      pl.BlockSpec((1, D_FULL), lambda i: (0, 0)),
  ]
  if split_out:
    out_specs = [pl.BlockSpec((R_BLK, HALF), lambda i: (i, 0))] * 2
    out_shape = [jax.ShapeDtypeStruct((N_NODES, HALF), jnp.float32)] * 2
  else:
    out_specs = pl.BlockSpec((R_BLK, D_FULL), lambda i: (i, 0))
    out_shape = jax.ShapeDtypeStruct((N_NODES, D_FULL), jnp.float32)
  return pl.pallas_call(body, grid=(NB,), in_specs=in_specs,
                        out_specs=out_specs, out_shape=out_shape)


_tc_layer1 = _make_tc_layer(relu=True, split_out=True)
_tc_layer2 = _make_tc_layer(relu=False, split_out=False)


def _unwrap(r):
  return r[0] if isinstance(r, (tuple, list)) else r


def kernel(x, edge_index, edge_attr, W1_l, b1, W1_r, W2_l, b2, W2_r):
  del edge_attr
  src = edge_index[0].astype(jnp.int32)
  dst = edge_index[1].astype(jnp.int32).reshape(NTEC, CHUNKS, K)
  src2 = jnp.stack([src, src + N_NODES]).reshape(2, NTEC, CHUNKS, K)

  xcat = jnp.concatenate([x[:, :HALF], x[:, HALF:]], axis=0)

  agg1 = _unwrap(_sc_agg(xcat, src2, dst))
  cnt = _unwrap(_sc_cnt(dst))

  w1ll, w1lh = W1_l.T[:HALF], W1_l.T[HALF:]
  w1rl, w1rh = W1_r.T[:HALF], W1_r.T[HALF:]
  h_lo, h_hi = _tc_layer1(agg1, agg1, cnt, xcat, xcat, w1ll, w1lh, w1rl,
                          w1rh, b1.reshape(1, D_FULL))

  hcat = jnp.concatenate([h_lo, h_hi], axis=0)
  agg2 = _unwrap(_sc_agg(hcat, src2, dst))

  w2ll, w2lh = W2_l.T[:HALF], W2_l.T[HALF:]
  w2rl, w2rh = W2_r.T[:HALF], W2_r.T[HALF:]
  out = _tc_layer2(agg2, agg2, cnt, hcat, hcat, w2ll, w2lh, w2rl, w2rh,
                   b2.reshape(1, D_FULL))
  return out
